# baseline (speedup 1.0000x reference)
import jax
import jax.numpy as jnp
from jax import lax
from jax.experimental import pallas as pl
from jax.experimental.pallas import tpu as pltpu
from jax.experimental.pallas import tpu_sc as plsc

EMBED_DIM = 64
BATCH = 16384
N_FIELDS = 26
NUM_FEAT = 1000000

NUM_CORES = 2
NUM_SUBCORES = 16
NUM_WORKERS = NUM_CORES * NUM_SUBCORES

FB = 512
N_BANDS = (NUM_FEAT + FB - 1) // FB

CB = 128
N_BLOCKS = N_FIELDS * (BATCH // CB)
BLOCKS_PER_WORKER = N_BLOCKS // NUM_WORKERS
B_ROUNDS = BLOCKS_PER_WORKER // 2
TCOLS = BATCH // CB


def _overlap_tc(wt_ref, out_ref):
    eye = jnp.eye(EMBED_DIM, dtype=jnp.float32)
    out_ref[:, :EMBED_DIM] = jax.lax.dot_general(
        wt_ref[...], eye, (((0,), (0,)), ((), ())),
        preferred_element_type=jnp.float32)


def _gather_body(wo_hbm, idx_hbm, out_hbm,
                 idx_all, rows0, rows1, g0, g1, s0, s1):
    wid = lax.axis_index("s") * NUM_CORES + lax.axis_index("c")

    pltpu.sync_copy(idx_hbm.at[pl.ds(wid * BLOCKS_PER_WORKER,
                                     BLOCKS_PER_WORKER), :], idx_all)

    def round_step(t, carry):
        t0 = 2 * t
        t1 = 2 * t + 1
        k0 = wid * BLOCKS_PER_WORKER + t0
        k1 = wid * BLOCKS_PER_WORKER + t1
        f0 = k0 // TCOLS
        tc0 = k0 % TCOLS
        f1 = k1 // TCOLS
        tc1 = k1 % TCOLS

        c0 = pltpu.async_copy(wo_hbm.at[idx_all.at[t0]], rows0, g0)
        c1 = pltpu.async_copy(wo_hbm.at[idx_all.at[t1]], rows1, g1)
        c0.wait()
        w0 = pltpu.async_copy(rows0, out_hbm.at[pl.ds(tc0 * CB, CB), f0, :], s0)
        c1.wait()
        w1 = pltpu.async_copy(rows1, out_hbm.at[pl.ds(tc1 * CB, CB), f1, :], s1)
        w0.wait()
        w1.wait()
        return carry

    lax.fori_loop(0, B_ROUNDS, round_step, 0)


def kernel(input_indices, W):
    wt = W.T
    idx2d = input_indices.T.astype(jnp.int32).reshape(N_BLOCKS, CB)
    mesh = plsc.VectorSubcoreMesh(core_axis_name="c", subcore_axis_name="s")

    w_over = pl.pallas_call(
        _overlap_tc,
        grid=(N_BANDS,),
        in_specs=[
            pl.BlockSpec((EMBED_DIM, FB), lambda i: (0, i)),
        ],
        out_specs=pl.BlockSpec((FB, 2 * EMBED_DIM), lambda i: (i, 0)),
        out_shape=jax.ShapeDtypeStruct((NUM_FEAT, 2 * EMBED_DIM),
                                       jnp.float32),
        compiler_params=pltpu.CompilerParams(
            dimension_semantics=("arbitrary",)),
    )(wt)

    out4 = pl.kernel(
        _gather_body,
        out_type=jax.ShapeDtypeStruct((BATCH, 32, 2 * EMBED_DIM),
                                      jnp.float32),
        mesh=mesh,
        scratch_types=[
            pltpu.VMEM((BLOCKS_PER_WORKER, CB), jnp.int32),
            pltpu.VMEM((CB, 2 * EMBED_DIM), jnp.float32),
            pltpu.VMEM((CB, 2 * EMBED_DIM), jnp.float32),
            pltpu.SemaphoreType.DMA,
            pltpu.SemaphoreType.DMA,
            pltpu.SemaphoreType.DMA,
            pltpu.SemaphoreType.DMA,
        ],
    )(w_over, idx2d)

    return out4[:, :N_FIELDS, :EMBED_DIM]


# speedup vs baseline: 1.9438x; 1.9438x over previous
import jax
import jax.numpy as jnp
from jax import lax
from jax.experimental import pallas as pl
from jax.experimental.pallas import tpu as pltpu
from jax.experimental.pallas import tpu_sc as plsc

EMBED_DIM = 64
BATCH = 16384
N_FIELDS = 26
NUM_FEAT = 1000000

NUM_CORES = 2
NUM_SUBCORES = 16
NUM_WORKERS = NUM_CORES * NUM_SUBCORES

FB = 512
N_BANDS = (NUM_FEAT + FB - 1) // FB

CB = 128
N_BLOCKS = N_FIELDS * (BATCH // CB)
BLOCKS_PER_WORKER = N_BLOCKS // NUM_WORKERS
B_ROUNDS = BLOCKS_PER_WORKER // 2
TCOLS = BATCH // CB


def _gather_body(wo_hbm, idx_hbm, out_hbm,
                 idx_all, rows0, rows1, g0, g1, s0, s1):
    wid = lax.axis_index("s") * NUM_CORES + lax.axis_index("c")

    pltpu.sync_copy(idx_hbm.at[pl.ds(wid * BLOCKS_PER_WORKER,
                                     BLOCKS_PER_WORKER), :], idx_all)

    def round_step(t, carry):
        t0 = 2 * t
        t1 = 2 * t + 1
        k0 = wid * BLOCKS_PER_WORKER + t0
        k1 = wid * BLOCKS_PER_WORKER + t1
        f0 = k0 // TCOLS
        tc0 = k0 % TCOLS
        f1 = k1 // TCOLS
        tc1 = k1 % TCOLS

        c0 = pltpu.async_copy(wo_hbm.at[idx_all.at[t0]], rows0, g0)
        c1 = pltpu.async_copy(wo_hbm.at[idx_all.at[t1]], rows1, g1)
        c0.wait()
        w0 = pltpu.async_copy(
            rows0, out_hbm.at[pl.ds(tc0 * CB, CB), f0, pl.ds(0, EMBED_DIM)], s0)
        c1.wait()
        w1 = pltpu.async_copy(
            rows1, out_hbm.at[pl.ds(tc1 * CB, CB), f1, pl.ds(0, EMBED_DIM)], s1)
        w0.wait()
        w1.wait()
        return carry

    lax.fori_loop(0, B_ROUNDS, round_step, 0)


def kernel(input_indices, W):
    idx2d = input_indices.T.astype(jnp.int32).reshape(N_BLOCKS, CB)
    mesh = plsc.VectorSubcoreMesh(core_axis_name="c", subcore_axis_name="s")

    out4 = pl.kernel(
        _gather_body,
        out_type=jax.ShapeDtypeStruct((BATCH, 32, 2 * EMBED_DIM),
                                      jnp.float32),
        mesh=mesh,
        scratch_types=[
            pltpu.VMEM((BLOCKS_PER_WORKER, CB), jnp.int32),
            pltpu.VMEM((CB, EMBED_DIM), jnp.float32),
            pltpu.VMEM((CB, EMBED_DIM), jnp.float32),
            pltpu.SemaphoreType.DMA,
            pltpu.SemaphoreType.DMA,
            pltpu.SemaphoreType.DMA,
            pltpu.SemaphoreType.DMA,
        ],
        compiler_params=pltpu.CompilerParams(use_tc_tiling_on_sc=False),
    )(W, idx2d)

    return out4[:, :N_FIELDS, :EMBED_DIM]


# speedup vs baseline: 1.9801x; 1.0187x over previous
import jax
import jax.numpy as jnp
from jax import lax
from jax.experimental import pallas as pl
from jax.experimental.pallas import tpu as pltpu
from jax.experimental.pallas import tpu_sc as plsc

EMBED_DIM = 64
BATCH = 16384
N_FIELDS = 26
NUM_FEAT = 1000000

NUM_CORES = 2
NUM_SUBCORES = 16
NUM_WORKERS = NUM_CORES * NUM_SUBCORES

FB = 512
N_BANDS = (NUM_FEAT + FB - 1) // FB

CB = 128
N_BLOCKS = N_FIELDS * (BATCH // CB)
BLOCKS_PER_WORKER = N_BLOCKS // NUM_WORKERS
B_ROUNDS = BLOCKS_PER_WORKER // 4
TCOLS = BATCH // CB


def _gather_body(wo_hbm, idx_hbm, out_hbm,
                 idx_all, rows0, rows1, rows2, rows3,
                 g0, g1, g2, g3, s0, s1, s2, s3):
    wid = lax.axis_index("s") * NUM_CORES + lax.axis_index("c")

    pltpu.sync_copy(idx_hbm.at[pl.ds(wid * BLOCKS_PER_WORKER,
                                     BLOCKS_PER_WORKER), :], idx_all)

    rows = (rows0, rows1, rows2, rows3)
    gsem = (g0, g1, g2, g3)
    ssem = (s0, s1, s2, s3)

    def round_step(t, carry):
        gathers = []
        for i in range(4):
            ti = 4 * t + i
            gathers.append(pltpu.async_copy(
                wo_hbm.at[idx_all.at[ti]], rows[i], gsem[i]))
        stores = []
        for i in range(4):
            ki = wid * BLOCKS_PER_WORKER + 4 * t + i
            fi = ki // TCOLS
            tci = ki % TCOLS
            gathers[i].wait()
            stores.append(pltpu.async_copy(
                rows[i],
                out_hbm.at[pl.ds(tci * CB, CB), fi, pl.ds(0, EMBED_DIM)],
                ssem[i]))
        for w in stores:
            w.wait()
        return carry

    lax.fori_loop(0, B_ROUNDS, round_step, 0)


def kernel(input_indices, W):
    idx2d = input_indices.T.astype(jnp.int32).reshape(N_BLOCKS, CB)
    mesh = plsc.VectorSubcoreMesh(core_axis_name="c", subcore_axis_name="s")

    out4 = pl.kernel(
        _gather_body,
        out_type=jax.ShapeDtypeStruct((BATCH, 32, 2 * EMBED_DIM),
                                      jnp.float32),
        mesh=mesh,
        scratch_types=[
            pltpu.VMEM((BLOCKS_PER_WORKER, CB), jnp.int32),
            pltpu.VMEM((CB, EMBED_DIM), jnp.float32),
            pltpu.VMEM((CB, EMBED_DIM), jnp.float32),
            pltpu.VMEM((CB, EMBED_DIM), jnp.float32),
            pltpu.VMEM((CB, EMBED_DIM), jnp.float32),
            pltpu.SemaphoreType.DMA,
            pltpu.SemaphoreType.DMA,
            pltpu.SemaphoreType.DMA,
            pltpu.SemaphoreType.DMA,
            pltpu.SemaphoreType.DMA,
            pltpu.SemaphoreType.DMA,
            pltpu.SemaphoreType.DMA,
            pltpu.SemaphoreType.DMA,
        ],
        compiler_params=pltpu.CompilerParams(use_tc_tiling_on_sc=False),
    )(W, idx2d)

    return out4[:, :N_FIELDS, :EMBED_DIM]
